# Initial kernel scaffold; baseline (speedup 1.0000x reference)
import jax
import jax.numpy as jnp
from jax.experimental import pallas as pl


def kernel(input_x, embedding_matrix):
    raise NotImplementedError("write your pallas kernel here")


# speedup vs baseline: 1.5809x; 1.5809x over previous
import functools

import jax
import jax.numpy as jnp
from jax import lax
from jax.experimental import pallas as pl
from jax.experimental.pallas import tpu as pltpu
from jax.experimental.pallas import tpu_sc as plsc

BATCH = 16384
FIELDS = 26
EMBED_DIM = 32
N_TOTAL = BATCH * FIELDS

NUM_CORES = 2
NUM_SUBCORES = 16
NW = NUM_CORES * NUM_SUBCORES
N_PER_W = N_TOTAL // NW
CHUNK = 1024
N_CHUNKS = N_PER_W // CHUNK

_mesh = plsc.VectorSubcoreMesh(core_axis_name="c", subcore_axis_name="s")


@functools.partial(
    pl.kernel,
    out_type=jax.ShapeDtypeStruct((N_TOTAL, EMBED_DIM), jnp.float32),
    mesh=_mesh,
    scratch_types=[
        pltpu.VMEM((N_PER_W,), jnp.int32),
        pltpu.VMEM((2, CHUNK, EMBED_DIM), jnp.float32),
        pltpu.SemaphoreType.DMA((2,)),
    ],
    compiler_params=pltpu.CompilerParams(use_tc_tiling_on_sc=False),
)
def _gather_kernel(idx_hbm, table_hbm, out_hbm, idx_v, rows_v, sems):
    wid = lax.axis_index("s") * NUM_CORES + lax.axis_index("c")
    base = wid * N_PER_W
    pltpu.sync_copy(idx_hbm.at[wid], idx_v)

    cps = [None, None]
    cps[0] = pltpu.async_copy(table_hbm.at[idx_v.at[pl.ds(0, CHUNK)]], rows_v.at[0], sems.at[0])
    for j in range(N_CHUNKS):
        slot = j % 2
        if j + 1 < N_CHUNKS:
            nslot = (j + 1) % 2
            cps[nslot] = pltpu.async_copy(
                table_hbm.at[idx_v.at[pl.ds((j + 1) * CHUNK, CHUNK)]], rows_v.at[nslot], sems.at[nslot]
            )
        cps[slot].wait()
        pltpu.sync_copy(rows_v.at[slot], out_hbm.at[pl.ds(base + j * CHUNK, CHUNK)])


def kernel(input_x, embedding_matrix):
    idx = input_x.reshape(NW, N_PER_W)
    out = _gather_kernel(idx, embedding_matrix)
    return out.reshape(BATCH, FIELDS, EMBED_DIM)
